# baseline (speedup 1.0000x reference)
import jax
import jax.numpy as jnp
from jax.experimental import pallas as pl

ROWS = 4096
COLS = 26
DEPTH = 1000
BLOCK_R = 128


def _one_hot_body(idx_ref, out_ref):
    idx = idx_ref[...]
    iota = jax.lax.broadcasted_iota(jnp.int32, (BLOCK_R, COLS, DEPTH), 2)
    out_ref[...] = (idx[:, :, None] == iota).astype(jnp.float32)


def kernel(inputs):
    idx = inputs.astype(jnp.int32)
    return pl.pallas_call(
        _one_hot_body,
        grid=(ROWS // BLOCK_R,),
        in_specs=[pl.BlockSpec((BLOCK_R, COLS), lambda i: (i, 0))],
        out_specs=pl.BlockSpec((BLOCK_R, COLS, DEPTH), lambda i: (i, 0, 0)),
        out_shape=jax.ShapeDtypeStruct((ROWS, COLS, DEPTH), jnp.float32),
    )(idx)
